# baseline (speedup 1.0000x reference)
import functools

import jax
import jax.numpy as jnp
from jax import lax
from jax.experimental import pallas as pl
from jax.experimental.pallas import tpu as pltpu
from jax.experimental.pallas import tpu_sc as plsc


def _matvec_body(t_ref, w_ref, o_ref):
    o_ref[...] = lax.dot_general(
        t_ref[...], w_ref[...],
        dimension_numbers=(((1,), (1,)), ((), ())),
        preferred_element_type=jnp.float32,
    )


def _table_dot_w(table, W):
    V, D = table.shape
    VB = 8000
    grid = V // VB
    return pl.pallas_call(
        _matvec_body,
        grid=(grid,),
        in_specs=[
            pl.BlockSpec((VB, D), lambda i: (i, 0)),
            pl.BlockSpec((1, D), lambda i: (0, 0)),
        ],
        out_specs=pl.BlockSpec((VB, 1), lambda i: (i, 0)),
        out_shape=jax.ShapeDtypeStruct((V, 1), jnp.float32),
    )(table, W)


def _make_sc_pool(VP, L, B):
    info = plsc.get_sparse_core_info()
    NC, NS = info.num_cores, info.num_subcores
    H = VP // NC
    BCOLS = B // NS
    CHUNK = 128
    NCHUNK = BCOLS // CHUNK
    NSLOT = 8
    NGROUP = L // NSLOT
    NVEC = CHUNK // 16
    STG = H // NS // 2

    mesh = plsc.VectorSubcoreMesh(core_axis_name="c", subcore_axis_name="s")

    @functools.partial(
        pl.kernel,
        mesh=mesh,
        out_type=jax.ShapeDtypeStruct((NC, B), jnp.float32),
        scratch_types=[
            pltpu.VMEM((L, CHUNK), jnp.int32),
            pltpu.VMEM((L * CHUNK,), jnp.int32),
            pltpu.VMEM((L * CHUNK,), jnp.float32),
            pltpu.VMEM((BCOLS,), jnp.float32),
            pltpu.VMEM((16,), jnp.float32),
            pltpu.VMEM((STG,), jnp.float32),
            pltpu.VMEM_SHARED((H + 16,), jnp.float32),
            pltpu.SemaphoreType.DMA,
        ],
    )
    def sc_pool(tw_hbm, x_hbm, out_hbm,
                idx_v, idxe_v, vals_v, out_v, zero_v, bounce, tw_sh, sem):
        cid = lax.axis_index("c")
        sid = lax.axis_index("s")
        base = sid * BCOLS
        coff = cid * H

        for k in range(2):
            off = sid * (2 * STG) + k * STG
            pltpu.sync_copy(tw_hbm.at[pl.ds(coff + off, STG)], bounce)
            pltpu.sync_copy(bounce, tw_sh.at[pl.ds(off, STG)])
        zero_v[...] = jnp.zeros((16,), jnp.float32)

        @pl.when(sid == 0)
        def _sentinel():
            pltpu.sync_copy(zero_v, tw_sh.at[pl.ds(H, 16)])

        plsc.subcore_barrier()

        hvec = jnp.full((16,), H, dtype=jnp.uint32)
        cvec = jnp.broadcast_to(coff.astype(jnp.int32), (16,))

        for c in range(NCHUNK):
            pltpu.sync_copy(x_hbm.at[:, pl.ds(base + c * CHUNK, CHUNK)], idx_v)

            def remap(g, carry):
                for r in range(NSLOT):
                    l = g * NSLOT + r
                    for j in range(NVEC):
                        v = idx_v[l, pl.ds(j * 16, 16)]
                        t = lax.bitcast_convert_type(v - cvec, jnp.uint32)
                        ie = jnp.minimum(t, hvec)
                        idxe_v[pl.ds(l * CHUNK + j * 16, 16)] = (
                            lax.bitcast_convert_type(ie, jnp.int32))
                return carry

            lax.fori_loop(0, NGROUP, remap, 0)

            pltpu.async_copy(tw_sh.at[idxe_v], vals_v, sem).wait()

            def reduce(g, ss):
                out = []
                for j in range(NVEC):
                    s = ss[j]
                    for r in range(NSLOT):
                        l = g * NSLOT + r
                        s = s + vals_v[pl.ds(l * CHUNK + j * 16, 16)]
                    out.append(s)
                return tuple(out)

            zeros = tuple(jnp.zeros((16,), jnp.float32) for _ in range(NVEC))
            sums = lax.fori_loop(0, NGROUP, reduce, zeros)

            for j in range(NVEC):
                out_v[pl.ds(c * CHUNK + j * 16, 16)] = sums[j]

        pltpu.sync_copy(out_v, out_hbm.at[cid, pl.ds(base, BCOLS)])

    return sc_pool


def _epilogue_body(p0_ref, p1_ref, len_ref, b_ref, o_ref):
    z = (p0_ref[...] + p1_ref[...]) / len_ref[0] + b_ref[0]
    o_ref[...] = jax.nn.sigmoid(z)


def _epilogue(p0, p1, lengths, b):
    R, C = p0.shape
    return pl.pallas_call(
        _epilogue_body,
        in_specs=[
            pl.BlockSpec((R, C), lambda: (0, 0)),
            pl.BlockSpec((R, C), lambda: (0, 0)),
            pl.BlockSpec(memory_space=pltpu.SMEM),
            pl.BlockSpec(memory_space=pltpu.SMEM),
        ],
        out_specs=pl.BlockSpec((R, C), lambda: (0, 0)),
        out_shape=jax.ShapeDtypeStruct((R, C), jnp.float32),
    )(p0, p1, lengths, b)


def kernel(x, lengths, table, W, b):
    L, B = x.shape
    V, D = table.shape
    x = x.astype(jnp.int32)

    tw = _table_dot_w(table, W).reshape(-1)
    VP = 1 << 20
    tw = jnp.pad(tw, (0, VP - V))

    partial = _make_sc_pool(VP, L, B)(tw, x)
    p0 = partial[0].reshape(128, B // 128)
    p1 = partial[1].reshape(128, B // 128)
    out = _epilogue(p0, p1, lengths, b)
    return out.reshape(B, 1)


# speedup vs baseline: 1.9087x; 1.9087x over previous
import functools

import jax
import jax.numpy as jnp
from jax import lax
from jax.experimental import pallas as pl
from jax.experimental.pallas import tpu as pltpu
from jax.experimental.pallas import tpu_sc as plsc


def _matvec_body(t_ref, w_ref, o_ref):
    o_ref[...] = lax.dot_general(
        t_ref[...], w_ref[...],
        dimension_numbers=(((1,), (1,)), ((), ())),
        preferred_element_type=jnp.float32,
    )


def _table_dot_w(table, W):
    V, D = table.shape
    VB = 8000
    grid = V // VB
    return pl.pallas_call(
        _matvec_body,
        grid=(grid,),
        in_specs=[
            pl.BlockSpec((VB, D), lambda i: (i, 0)),
            pl.BlockSpec((1, D), lambda i: (0, 0)),
        ],
        out_specs=pl.BlockSpec((VB, 1), lambda i: (i, 0)),
        out_shape=jax.ShapeDtypeStruct((V, 1), jnp.float32),
    )(table, W)


def _make_sc_pool(VP, L, B):
    info = plsc.get_sparse_core_info()
    NC, NS = info.num_cores, info.num_subcores
    H = VP // NC
    BCOLS = B // NS
    CHUNK = 128
    NCHUNK = BCOLS // CHUNK
    NSLOT = 8
    NGROUP = L // NSLOT
    NVEC = CHUNK // 16
    NSTG = 8
    STG = H // NS // NSTG
    SPAD = 2048

    mesh = plsc.VectorSubcoreMesh(core_axis_name="c", subcore_axis_name="s")

    @functools.partial(
        pl.kernel,
        mesh=mesh,
        out_type=jax.ShapeDtypeStruct((NC, B), jnp.float32),
        scratch_types=[
            pltpu.VMEM((L, CHUNK), jnp.int32),
            pltpu.VMEM((L * CHUNK,), jnp.int32),
            pltpu.VMEM((L * CHUNK,), jnp.float32),
            pltpu.VMEM((BCOLS,), jnp.float32),
            pltpu.VMEM((STG,), jnp.float32),
            pltpu.VMEM_SHARED((H + SPAD,), jnp.float32),
            pltpu.SemaphoreType.DMA,
        ],
    )
    def sc_pool(tw_hbm, x_hbm, out_hbm,
                idx_v, idxe_v, vals_v, out_v, bounce, tw_sh, sem):
        cid = lax.axis_index("c")
        sid = lax.axis_index("s")
        base = sid * BCOLS
        coff = cid * H

        def stage(k, carry):
            off = sid * (NSTG * STG) + k * STG
            pltpu.sync_copy(tw_hbm.at[pl.ds(coff + off, STG)], bounce)
            pltpu.sync_copy(bounce, tw_sh.at[pl.ds(off, STG)])
            return carry

        lax.fori_loop(0, NSTG, stage, 0)

        @pl.when(sid == 0)
        def _sentinel():
            pltpu.sync_copy(tw_hbm.at[pl.ds(VP - SPAD, SPAD)],
                            bounce.at[pl.ds(0, SPAD)])
            pltpu.sync_copy(bounce.at[pl.ds(0, SPAD)], tw_sh.at[pl.ds(H, SPAD)])

        plsc.subcore_barrier()

        hvec = jnp.full((16,), H, dtype=jnp.uint32)
        hivec = jnp.full((16,), H, dtype=jnp.int32)
        smask = jnp.full((16,), SPAD - 1, dtype=jnp.int32)
        cvec = jnp.broadcast_to(coff.astype(jnp.int32), (16,))

        for c in range(NCHUNK):
            pltpu.sync_copy(x_hbm.at[:, pl.ds(base + c * CHUNK, CHUNK)], idx_v)

            def remap(g, carry):
                for r in range(4):
                    l = g * 4 + r
                    for j in range(NVEC):
                        v = idx_v[l, pl.ds(j * 16, 16)]
                        loc = v - cvec
                        oob = lax.bitcast_convert_type(loc, jnp.uint32) >= hvec
                        alt = (v & smask) + hivec
                        idxe_v[pl.ds(l * CHUNK + j * 16, 16)] = (
                            jnp.where(oob, alt, loc))
                return carry

            lax.fori_loop(0, L // 4, remap, 0)

            pltpu.async_copy(tw_sh.at[idxe_v], vals_v, sem).wait()

            def reduce(g, ss):
                out = []
                for j in range(NVEC):
                    s = ss[j]
                    for r in range(NSLOT):
                        l = g * NSLOT + r
                        s = s + vals_v[pl.ds(l * CHUNK + j * 16, 16)]
                    out.append(s)
                return tuple(out)

            zeros = tuple(jnp.zeros((16,), jnp.float32) for _ in range(NVEC))
            sums = lax.fori_loop(0, NGROUP, reduce, zeros)

            for j in range(NVEC):
                out_v[pl.ds(c * CHUNK + j * 16, 16)] = sums[j]

        pltpu.sync_copy(out_v, out_hbm.at[cid, pl.ds(base, BCOLS)])

    return sc_pool


def _epilogue_body(p0_ref, p1_ref, len_ref, b_ref, o_ref):
    z = (p0_ref[...] + p1_ref[...]) / len_ref[0] + b_ref[0]
    o_ref[...] = jax.nn.sigmoid(z)


def _epilogue(p0, p1, lengths, b):
    R, C = p0.shape
    return pl.pallas_call(
        _epilogue_body,
        in_specs=[
            pl.BlockSpec((R, C), lambda: (0, 0)),
            pl.BlockSpec((R, C), lambda: (0, 0)),
            pl.BlockSpec(memory_space=pltpu.SMEM),
            pl.BlockSpec(memory_space=pltpu.SMEM),
        ],
        out_specs=pl.BlockSpec((R, C), lambda: (0, 0)),
        out_shape=jax.ShapeDtypeStruct((R, C), jnp.float32),
    )(p0, p1, lengths, b)


def kernel(x, lengths, table, W, b):
    L, B = x.shape
    V, D = table.shape
    x = x.astype(jnp.int32)

    tw = _table_dot_w(table, W).reshape(-1)
    VP = 1 << 20
    tw = jnp.pad(tw, (0, VP - V))

    partial = _make_sc_pool(VP, L, B)(tw, x)
    p0 = partial[0].reshape(128, B // 128)
    p1 = partial[1].reshape(128, B // 128)
    out = _epilogue(p0, p1, lengths, b)
    return out.reshape(B, 1)
